# baseline (speedup 1.0000x reference)
import functools

import jax
import jax.numpy as jnp
from jax import lax
from jax.experimental import pallas as pl
from jax.experimental.pallas import tpu as pltpu
from jax.experimental.pallas import tpu_sc as plsc

NUM_FIELDS = 26
VOCAB = 100000
EMB_DIM = 32
BATCH = 16384
CHUNK = 4096


def _sc_gather_t(ids_t, tables_t):
    mesh = plsc.VectorSubcoreMesh(core_axis_name="c", subcore_axis_name="s")

    @functools.partial(
        pl.kernel,
        mesh=mesh,
        compiler_params=pltpu.CompilerParams(use_tc_tiling_on_sc=True,
                                             needs_layout_passes=False),
        out_type=jax.ShapeDtypeStruct((NUM_FIELDS * EMB_DIM, BATCH),
                                      jnp.float32),
        scratch_types=[
            pltpu.VMEM((VOCAB,), jnp.float32),
            pltpu.VMEM((2, CHUNK), jnp.int32),
            pltpu.VMEM((2, 2 * CHUNK), jnp.float32),
            pltpu.VMEM_SHARED((2, BATCH), jnp.int32),
            pltpu.SemaphoreType.DMA,
            pltpu.SemaphoreType.DMA,
            pltpu.SemaphoreType.DMA,
        ],
    )
    def k(ids_hbm, tab_hbm, out_hbm, row_v, ids_v, out_v, ids_sh, sem_row,
          sem_ids, sem_out):
        d = lax.axis_index("c") * 16 + lax.axis_index("s")
        nchunk = BATCH // CHUNK

        @pl.when(lax.axis_index("s") == 0)
        def _stage0():
            pltpu.sync_copy(ids_hbm.at[0], ids_sh.at[0])

        pltpu.async_copy(tab_hbm.at[0, d], row_v, sem_row)
        plsc.subcore_barrier()

        def body(j, carry):
            jj = lax.rem(j, 2)

            @pl.when(jnp.logical_and(lax.axis_index("s") == 0,
                                     j < NUM_FIELDS - 1))
            def _stage_next():
                pltpu.sync_copy(ids_hbm.at[j + 1], ids_sh.at[lax.rem(j + 1, 2)])

            pltpu.async_copy(ids_sh.at[jj, pl.ds(0, CHUNK)],
                             ids_v.at[0], sem_ids).wait()
            pltpu.make_async_copy(tab_hbm.at[j, d], row_v, sem_row).wait()
            out_cps = [None, None]
            for c in range(nchunk):
                if c + 1 < nchunk:
                    nxt = pltpu.async_copy(
                        ids_sh.at[jj, pl.ds((c + 1) * CHUNK, CHUNK)],
                        ids_v.at[(c + 1) % 2], sem_ids)

                @plsc.parallel_loop(0, CHUNK // 16, unroll=16)
                def gather16(i, c=c):
                    idx = ids_v[(c % 2), pl.ds(i * 16, 16)]
                    out_v[(c // 2), pl.ds((c % 2) * CHUNK + i * 16, 16)] = (
                        plsc.load_gather(row_v, [idx]))

                if c % 2 == 1:
                    out_cps[c // 2] = pltpu.async_copy(
                        out_v.at[c // 2],
                        out_hbm.at[j * EMB_DIM + d,
                                   pl.ds((c // 2) * 2 * CHUNK, 2 * CHUNK)],
                        sem_out)
                if c + 1 < nchunk:
                    nxt.wait()
                if c == nchunk - 1:
                    @pl.when(j < NUM_FIELDS - 1)
                    def _next_row():
                        pltpu.async_copy(tab_hbm.at[j + 1, d], row_v, sem_row)

            plsc.subcore_barrier()
            out_cps[0].wait()
            out_cps[1].wait()
            return carry

        lax.fori_loop(0, NUM_FIELDS, body, 0)

    return k(ids_t, tables_t)


def kernel(x_cat_ids, tables):
    ids_t = x_cat_ids.T.astype(jnp.int32)
    tables_t = jnp.transpose(tables, (0, 2, 1))
    out_t = _sc_gather_t(ids_t, tables_t)
    return out_t.T
